# Initial kernel scaffold; baseline (speedup 1.0000x reference)
import jax
import jax.numpy as jnp
from jax import lax
from jax.experimental import pallas as pl
from jax.experimental.pallas import tpu as pltpu
from jax.experimental.pallas import tpu_sc as plsc

_HU = (-1000.0, -900.0, -400.0, -100.0, -50.0, -10.0,
       20.0, 40.0, 60.0, 100.0, 800.0, 1000.0)

_NROW = 512
_NQ = 4096
_GQ = 16
_NGROUP = _NQ // _GQ
_NTILE = 32
_GP_PER_TILE = _NGROUP // _NTILE
_OUT_PER_GROUP = _GQ * 8 * _NROW
_HALF = _OUT_PER_GROUP // 2


def _sc_body(x_hbm, tab_hbm, out_hbm, xbuf, binsT, obufA, obufB, tabv,
             semA, semB):
    cid = lax.axis_index("c")
    sid = lax.axis_index("s")
    wid = sid * 2 + cid

    pltpu.sync_copy(tab_hbm, tabv)
    lanes = lax.iota(jnp.int32, 16)
    scat_base = lanes * _NROW

    pending = [None, None]
    for gi in range(_GP_PER_TILE):
        grp = wid * _GP_PER_TILE + gi
        e = grp // 16
        f = grp % 16
        pltpu.sync_copy(x_hbm.at[:, e, f, :], xbuf)

        def p1(i, carry):
            xr = xbuf[i, :]
            b = jnp.full((16,), 8, jnp.int32)
            for hu in _HU[1:]:
                b = b + jnp.where(xr >= hu, 8, 0)
            plsc.store_scatter(binsT, [scat_base + i], b)
            return carry

        lax.fori_loop(0, _NROW, p1, 0)

        for half, obuf, sem in ((0, obufA, semA), (1, obufB, semB)):
            if pending[half] is not None:
                pending[half].wait()

            def p2(u, carry, half=half, obuf=obuf):
                gl = u // 32
                j = u % 32
                b = binsT[pl.ds(half * 4096 + gl * 512 + j * 16, 16)]
                for vd in range(8):
                    t = plsc.load_gather(tabv, [b + vd])
                    obuf[pl.ds(gl * 4096 + vd * 512 + j * 16, 16)] = t
                return carry

            lax.fori_loop(0, 256, p2, 0)
            dst = out_hbm.at[pl.ds(grp * _OUT_PER_GROUP + half * _HALF,
                                   _HALF)]
            pending[half] = pltpu.async_copy(obuf, dst, sem)

    for p in pending:
        p.wait()


@jax.jit
def _run(x4, tab):
    mesh = plsc.VectorSubcoreMesh(core_axis_name="c", subcore_axis_name="s",
                                  num_cores=2, num_subcores=16)
    return pl.kernel(
        _sc_body,
        out_type=jax.ShapeDtypeStruct((_NROW * _NQ * 8,), jnp.float32),
        mesh=mesh,
        scratch_types=[
            pltpu.VMEM((_NROW, _GQ), jnp.float32),
            pltpu.VMEM((_GQ * _NROW,), jnp.int32),
            pltpu.VMEM((_HALF,), jnp.float32),
            pltpu.VMEM((_HALF,), jnp.float32),
            pltpu.VMEM((128,), jnp.float32),
            pltpu.SemaphoreType.DMA,
            pltpu.SemaphoreType.DMA,
        ],
    )(x4, tab)


def kernel(x, vectors):
    x4 = x.reshape(_NROW, _GQ, _GQ, _GQ)
    tab = jnp.concatenate(
        [vectors.reshape(-1).astype(jnp.float32),
         jnp.zeros((128 - vectors.size,), jnp.float32)])
    out = _run(x4, tab)
    return out.reshape(1, 32768, 8, 8, 8)


# speedup vs baseline: 2.5449x; 2.5449x over previous
import jax
import jax.numpy as jnp
from jax import lax
from jax.experimental import pallas as pl
from jax.experimental.pallas import tpu as pltpu
from jax.experimental.pallas import tpu_sc as plsc

_HU = (-1000.0, -900.0, -400.0, -100.0, -50.0, -10.0,
       20.0, 40.0, 60.0, 100.0, 800.0, 1000.0)

_NROW = 512
_NQ = 4096
_GQ = 16
_NGROUP = _NQ // _GQ
_NTILE = 32
_GP_PER_TILE = _NGROUP // _NTILE
_OUT_PER_GROUP = _GQ * 8 * _NROW
_QTR = _OUT_PER_GROUP // 4


def _sc_body(x_hbm, tab_hbm, out_hbm, xbuf, binsT, obufA, obufB, tabv,
             semA, semB):
    cid = lax.axis_index("c")
    sid = lax.axis_index("s")
    wid = sid * 2 + cid

    pltpu.sync_copy(tab_hbm, tabv)
    lanes = lax.iota(jnp.int32, 16)
    scat_base = lanes * _NROW

    pending = [None, None]
    for gi in range(_GP_PER_TILE):
        grp = wid * _GP_PER_TILE + gi
        e = grp // 16
        f = grp % 16
        pltpu.sync_copy(x_hbm.at[:, e, f, :], xbuf)

        def p1(i, carry):
            xr = xbuf[i, :]
            b = jnp.full((16,), 8, jnp.int32)
            for hu in _HU[1:]:
                b = b + jnp.where(xr >= hu, 8, 0)
            plsc.store_scatter(binsT, [scat_base + i], b)
            return carry

        lax.fori_loop(0, _NROW, p1, 0)

        for qt in range(4):
            buf_i = qt % 2
            obuf = (obufA, obufB)[buf_i]
            sem = (semA, semB)[buf_i]
            if pending[buf_i] is not None:
                pending[buf_i].wait()

            def p2(u, carry, qt=qt, obuf=obuf):
                gl = u // 32
                j = u % 32
                b = binsT[pl.ds((qt * 4 + gl) * 512 + j * 16, 16)]
                for vd in range(8):
                    t = plsc.load_gather(tabv, [b + vd])
                    obuf[pl.ds(gl * 4096 + vd * 512 + j * 16, 16)] = t
                return carry

            lax.fori_loop(0, 128, p2, 0)
            dst = out_hbm.at[pl.ds(grp * _OUT_PER_GROUP + qt * _QTR, _QTR)]
            pending[buf_i] = pltpu.async_copy(obuf, dst, sem)

    for p in pending:
        p.wait()


@jax.jit
def _run(x4, tab):
    mesh = plsc.VectorSubcoreMesh(core_axis_name="c", subcore_axis_name="s",
                                  num_cores=2, num_subcores=16)
    return pl.kernel(
        _sc_body,
        out_type=jax.ShapeDtypeStruct((_NROW * _NQ * 8,), jnp.float32),
        mesh=mesh,
        compiler_params=pltpu.CompilerParams(needs_layout_passes=False),
        scratch_types=[
            pltpu.VMEM((_NROW, _GQ), jnp.float32),
            pltpu.VMEM((_GQ * _NROW,), jnp.int32),
            pltpu.VMEM((_QTR,), jnp.float32),
            pltpu.VMEM((_QTR,), jnp.float32),
            pltpu.VMEM((128,), jnp.float32),
            pltpu.SemaphoreType.DMA,
            pltpu.SemaphoreType.DMA,
        ],
    )(x4, tab)


def kernel(x, vectors):
    x4 = x.reshape(_NROW, _GQ, _GQ, _GQ)
    tab = jnp.concatenate(
        [vectors.reshape(-1).astype(jnp.float32),
         jnp.zeros((128 - vectors.size,), jnp.float32)])
    out = _run(x4, tab)
    return out.reshape(1, 32768, 8, 8, 8)


# speedup vs baseline: 6.2563x; 2.4584x over previous
import jax
import jax.numpy as jnp
from jax import lax
from jax.experimental import pallas as pl
from jax.experimental.pallas import tpu as pltpu
from jax.experimental.pallas import tpu_sc as plsc

_HU = (-1000.0, -900.0, -400.0, -100.0, -50.0, -10.0,
       20.0, 40.0, 60.0, 100.0, 800.0, 1000.0)

_NROW = 512
_NQ = 4096
_NGROUP = 256
_QROW = 128


def _sc_body(x_hbm, tab_hbm, out_hbm, xq, binsQ, obufA, obufB, tabv,
             semA, semB):
    cid = lax.axis_index("c")
    sid = lax.axis_index("s")
    wid = sid * 2 + cid

    pltpu.sync_copy(tab_hbm, tabv)
    lanes = lax.iota(jnp.int32, 16)
    scat_g = lanes * _QROW
    row_pat = lanes // 8
    c_pat = lanes % 8
    zero16 = lanes * 0

    pending = [None, None]
    nbuf = 0
    for qt in range(4):
        pltpu.sync_copy(x_hbm.at[pl.ds(qt * _QROW, _QROW), wid, :], xq)

        def p1(i, carry):
            for gsub in range(8):
                xr = xq[i, pl.ds(gsub * 16, 16)]
                b = jnp.full((16,), 8, jnp.int32)
                for hu in _HU[1:]:
                    b = b + jnp.where(xr >= hu, 8, 0)
                plsc.store_scatter(binsQ, [gsub * 2048 + scat_g + i], b)
            return carry

        lax.fori_loop(0, _QROW, p1, 0)

        for gsub in range(8):
            buf_i = nbuf % 2
            nbuf += 1
            obuf = (obufA, obufB)[buf_i]
            sem = (semA, semB)[buf_i]
            if pending[buf_i] is not None:
                pending[buf_i].wait()

            def p2(u, carry, gsub=gsub, obuf=obuf):
                g = u // 8
                j8 = u % 8
                b = binsQ[pl.ds(gsub * 2048 + g * _QROW + j8 * 16, 16)]
                rows = row_pat + 2 * j8
                fl0 = g * 8
                for vd in range(8):
                    t = plsc.load_gather(tabv, [b + vd])
                    plsc.store_scatter(obuf, [rows, c_pat, zero16 + (fl0 + vd)], t)
                return carry

            lax.fori_loop(0, 128, p2, 0)
            grp = wid * 8 + gsub
            dst = out_hbm.at[pl.ds(qt * 16, 16), grp]
            pending[buf_i] = pltpu.async_copy(obuf, dst, sem)

    for p in pending:
        p.wait()


@jax.jit
def _run(x3, tab):
    mesh = plsc.VectorSubcoreMesh(core_axis_name="c", subcore_axis_name="s",
                                  num_cores=2, num_subcores=16)
    return pl.kernel(
        _sc_body,
        out_type=jax.ShapeDtypeStruct((64, _NGROUP, 8, 128), jnp.float32),
        mesh=mesh,
        compiler_params=pltpu.CompilerParams(needs_layout_passes=False),
        scratch_types=[
            pltpu.VMEM((_QROW, 128), jnp.float32),
            pltpu.VMEM((8 * 16 * _QROW,), jnp.int32),
            pltpu.VMEM((16, 8, 128), jnp.float32),
            pltpu.VMEM((16, 8, 128), jnp.float32),
            pltpu.VMEM((128,), jnp.float32),
            pltpu.SemaphoreType.DMA,
            pltpu.SemaphoreType.DMA,
        ],
    )(x3, tab)


def kernel(x, vectors):
    x3 = x.reshape(_NROW, 32, 128)
    tab = jnp.concatenate(
        [vectors.reshape(-1).astype(jnp.float32),
         jnp.zeros((128 - vectors.size,), jnp.float32)])
    out4 = _run(x3, tab)
    out6 = out4.reshape(1, 8, 8, _NGROUP, 8, 128)
    outT = jnp.transpose(out6, (0, 3, 5, 1, 2, 4))
    return outT.reshape(1, 32768, 8, 8, 8)


# speedup vs baseline: 11.2666x; 1.8008x over previous
import jax
import jax.numpy as jnp
from jax import lax
from jax.experimental import pallas as pl
from jax.experimental.pallas import tpu as pltpu
from jax.experimental.pallas import tpu_sc as plsc

_HU = (-1000.0, -900.0, -400.0, -100.0, -50.0, -10.0,
       20.0, 40.0, 60.0, 100.0, 800.0, 1000.0)

_NROW = 512
_NQ = 4096
_NGROUP = 256
_QROW = 128


def _sc_body(x_hbm, tab_hbm, out_hbm, xq, xq2, binsQ, obufA, obufB, tabv,
             semA, semB, semX):
    cid = lax.axis_index("c")
    sid = lax.axis_index("s")
    wid = sid * 2 + cid

    pltpu.sync_copy(tab_hbm, tabv)
    lanes = lax.iota(jnp.int32, 16)
    scat_g = lanes * _QROW
    row_pat = lanes // 8
    c_pat = lanes % 8
    zero16 = lanes * 0

    pending = [None, None]
    nbuf = 0
    xbufs = (xq, xq2)
    xcopy = pltpu.async_copy(x_hbm.at[pl.ds(0, _QROW), wid, :], xq, semX)
    for qt in range(4):
        xcopy.wait()
        xcur = xbufs[qt % 2]
        if qt < 3:
            xcopy = pltpu.async_copy(
                x_hbm.at[pl.ds((qt + 1) * _QROW, _QROW), wid, :],
                xbufs[(qt + 1) % 2], semX)

        for gsub in range(8):

            @plsc.parallel_loop(0, _QROW, unroll=2)
            def p1(i, gsub=gsub, xcur=xcur):
                xr = xcur[i, pl.ds(gsub * 16, 16)]
                b = zero16 + 1
                for hu in _HU[1:]:
                    b = b + (xr >= hu).astype(jnp.int32)
                plsc.store_scatter(binsQ, [scat_g + i], b * 8)

            buf_i = nbuf % 2
            nbuf += 1
            obuf = (obufA, obufB)[buf_i]
            sem = (semA, semB)[buf_i]
            if pending[buf_i] is not None:
                pending[buf_i].wait()

            @plsc.parallel_loop(0, 128, unroll=2)
            def p2(u, obuf=obuf):
                g = lax.shift_right_logical(u, 3)
                j8 = lax.bitwise_and(u, 7)
                b = binsQ[pl.ds(g * _QROW + j8 * 16, 16)]
                rows = row_pat + 2 * j8
                flv = zero16 + g * 8
                for vd in range(8):
                    t = plsc.load_gather(tabv, [b + vd])
                    plsc.store_scatter(obuf, [rows, c_pat, flv + vd], t)

            grp = wid * 8 + gsub
            dst = out_hbm.at[pl.ds(qt * 16, 16), grp]
            pending[buf_i] = pltpu.async_copy(obuf, dst, sem)

    for p in pending:
        p.wait()


@jax.jit
def _run(x3, tab):
    mesh = plsc.VectorSubcoreMesh(core_axis_name="c", subcore_axis_name="s",
                                  num_cores=2, num_subcores=16)
    return pl.kernel(
        _sc_body,
        out_type=jax.ShapeDtypeStruct((64, _NGROUP, 8, 128), jnp.float32),
        mesh=mesh,
        compiler_params=pltpu.CompilerParams(needs_layout_passes=False),
        scratch_types=[
            pltpu.VMEM((_QROW, 128), jnp.float32),
            pltpu.VMEM((_QROW, 128), jnp.float32),
            pltpu.VMEM((16 * _QROW,), jnp.int32),
            pltpu.VMEM((16, 8, 128), jnp.float32),
            pltpu.VMEM((16, 8, 128), jnp.float32),
            pltpu.VMEM((128,), jnp.float32),
            pltpu.SemaphoreType.DMA,
            pltpu.SemaphoreType.DMA,
            pltpu.SemaphoreType.DMA,
        ],
    )(x3, tab)


def kernel(x, vectors):
    x3 = x.reshape(_NROW, 32, 128)
    tab = jnp.concatenate(
        [vectors.reshape(-1).astype(jnp.float32),
         jnp.zeros((128 - vectors.size,), jnp.float32)])
    out4 = _run(x3, tab)
    out6 = out4.reshape(1, 8, 8, _NGROUP, 8, 128)
    outT = jnp.transpose(out6, (0, 3, 5, 1, 2, 4))
    return outT.reshape(1, 32768, 8, 8, 8)
